# baseline (speedup 1.0000x reference)
import jax
import jax.numpy as jnp
from jax.experimental import pallas as pl

L = 3


def _body(w0_ref, w1_ref, w2_ref, b0_ref, b1_ref, b2_ref,
          wet_ref, bet_ref, inpt_ref, outt_ref, outc_ref,
          ow0_ref, ow1_ref, ow2_ref, ob0_ref, ob1_ref, ob2_ref):
    we = wet_ref[0, 0, :]
    be = bet_ref[0, 0, :]
    inp_row = inpt_ref[0, 0, :]
    out_row = outt_ref[0, 0, :]
    out_col = outc_ref[0]

    ow0_ref[0, 0] = w0_ref[0, 0] + (we[0] + inp_row)[None, :]
    ow1_ref[0, 0] = w1_ref[0, 0] + we[1]
    ow2_ref[0, 0] = w2_ref[0, 0] + we[2] + out_col
    ob0_ref[0, 0] = b0_ref[0, 0] + be[0]
    ob1_ref[0, 0] = b1_ref[0, 0] + be[1]
    ob2_ref[0, 0] = b2_ref[0, 0] + (be[2] + out_row)[None, :]


def kernel(w0, w1, w2, b0, b1, b2, weight_emb, bias_emb, inp_emb, out_emb):
    B, C, H, NI = w0.shape
    NO = w2.shape[2]

    wet = weight_emb.T.reshape(C, 1, L)
    bet = bias_emb.T.reshape(C, 1, L)
    inpt = inp_emb.T.reshape(C, 1, NI)
    outt = out_emb.T.reshape(C, 1, NO)
    outc = out_emb.T.reshape(C, NO, 1)

    b0r = b0.reshape(B, C, 1, H)
    b1r = b1.reshape(B, C, 1, H)
    b2r = b2.reshape(B, C, 1, NO)

    bc = lambda b, c: (b, c, 0, 0)
    cc = lambda b, c: (c, 0, 0)

    out_shapes = (
        jax.ShapeDtypeStruct((B, C, H, NI), w0.dtype),
        jax.ShapeDtypeStruct((B, C, H, H), w1.dtype),
        jax.ShapeDtypeStruct((B, C, NO, H), w2.dtype),
        jax.ShapeDtypeStruct((B, C, 1, H), b0.dtype),
        jax.ShapeDtypeStruct((B, C, 1, H), b1.dtype),
        jax.ShapeDtypeStruct((B, C, 1, NO), b2.dtype),
    )
    in_specs = [
        pl.BlockSpec((1, 1, H, NI), bc),
        pl.BlockSpec((1, 1, H, H), bc),
        pl.BlockSpec((1, 1, NO, H), bc),
        pl.BlockSpec((1, 1, 1, H), bc),
        pl.BlockSpec((1, 1, 1, H), bc),
        pl.BlockSpec((1, 1, 1, NO), bc),
        pl.BlockSpec((1, 1, L), cc),
        pl.BlockSpec((1, 1, L), cc),
        pl.BlockSpec((1, 1, NI), cc),
        pl.BlockSpec((1, 1, NO), cc),
        pl.BlockSpec((1, NO, 1), cc),
    ]
    out_specs = (
        pl.BlockSpec((1, 1, H, NI), bc),
        pl.BlockSpec((1, 1, H, H), bc),
        pl.BlockSpec((1, 1, NO, H), bc),
        pl.BlockSpec((1, 1, 1, H), bc),
        pl.BlockSpec((1, 1, 1, H), bc),
        pl.BlockSpec((1, 1, 1, NO), bc),
    )

    ow0, ow1, ow2, ob0, ob1, ob2 = pl.pallas_call(
        _body,
        grid=(B, C),
        in_specs=in_specs,
        out_specs=out_specs,
        out_shape=out_shapes,
    )(w0, w1, w2, b0r, b1r, b2r, wet, bet, inpt, outt, outc)

    return (ow0, ow1, ow2,
            ob0.reshape(B, C, H), ob1.reshape(B, C, H), ob2.reshape(B, C, NO))


# speedup vs baseline: 1.1358x; 1.1358x over previous
import jax
import jax.numpy as jnp
from jax.experimental import pallas as pl

L = 3
CSPLIT = 2


def _body(w0_ref, w1_ref, w2_ref, b0_ref, b1_ref, b2_ref,
          wet_ref, bet_ref, inpt_ref, outt_ref, outc_ref,
          ow0_ref, ow1_ref, ow2_ref, ob0_ref, ob1_ref, ob2_ref):
    we0 = wet_ref[:, 0, 0]
    we1 = wet_ref[:, 0, 1]
    we2 = wet_ref[:, 0, 2]
    be0 = bet_ref[:, 0, 0]
    be1 = bet_ref[:, 0, 1]
    be2 = bet_ref[:, 0, 2]

    add0 = we0[:, None] + inpt_ref[:, 0, :]
    ow0_ref[0] = w0_ref[0] + add0[:, None, :]
    ow1_ref[0] = w1_ref[0] + we1[:, None, None]
    ow2_ref[0] = w2_ref[0] + we2[:, None, None] + outc_ref[...]
    ob0_ref[0] = b0_ref[0] + be0[:, None, None]
    ob1_ref[0] = b1_ref[0] + be1[:, None, None]
    ob2_ref[0] = b2_ref[0] + be2[:, None, None] + outt_ref[...]


def kernel(w0, w1, w2, b0, b1, b2, weight_emb, bias_emb, inp_emb, out_emb):
    B, C, H, NI = w0.shape
    NO = w2.shape[2]
    Cb = C // CSPLIT

    wet = weight_emb.T.reshape(C, 1, L)
    bet = bias_emb.T.reshape(C, 1, L)
    inpt = inp_emb.T.reshape(C, 1, NI)
    outt = out_emb.T.reshape(C, 1, NO)
    outc = out_emb.T.reshape(C, NO, 1)

    b0r = b0.reshape(B, C, 1, H)
    b1r = b1.reshape(B, C, 1, H)
    b2r = b2.reshape(B, C, 1, NO)

    bc = lambda i, j: (i, j, 0, 0)
    cc = lambda i, j: (j, 0, 0)

    out_shapes = (
        jax.ShapeDtypeStruct((B, C, H, NI), w0.dtype),
        jax.ShapeDtypeStruct((B, C, H, H), w1.dtype),
        jax.ShapeDtypeStruct((B, C, NO, H), w2.dtype),
        jax.ShapeDtypeStruct((B, C, 1, H), b0.dtype),
        jax.ShapeDtypeStruct((B, C, 1, H), b1.dtype),
        jax.ShapeDtypeStruct((B, C, 1, NO), b2.dtype),
    )
    in_specs = [
        pl.BlockSpec((1, Cb, H, NI), bc),
        pl.BlockSpec((1, Cb, H, H), bc),
        pl.BlockSpec((1, Cb, NO, H), bc),
        pl.BlockSpec((1, Cb, 1, H), bc),
        pl.BlockSpec((1, Cb, 1, H), bc),
        pl.BlockSpec((1, Cb, 1, NO), bc),
        pl.BlockSpec((Cb, 1, L), cc),
        pl.BlockSpec((Cb, 1, L), cc),
        pl.BlockSpec((Cb, 1, NI), cc),
        pl.BlockSpec((Cb, 1, NO), cc),
        pl.BlockSpec((Cb, NO, 1), cc),
    ]
    out_specs = (
        pl.BlockSpec((1, Cb, H, NI), bc),
        pl.BlockSpec((1, Cb, H, H), bc),
        pl.BlockSpec((1, Cb, NO, H), bc),
        pl.BlockSpec((1, Cb, 1, H), bc),
        pl.BlockSpec((1, Cb, 1, H), bc),
        pl.BlockSpec((1, Cb, 1, NO), bc),
    )

    ow0, ow1, ow2, ob0, ob1, ob2 = pl.pallas_call(
        _body,
        grid=(B, CSPLIT),
        in_specs=in_specs,
        out_specs=out_specs,
        out_shape=out_shapes,
    )(w0, w1, w2, b0r, b1r, b2r, wet, bet, inpt, outt, outc)

    return (ow0, ow1, ow2,
            ob0.reshape(B, C, H), ob1.reshape(B, C, H), ob2.reshape(B, C, NO))
